# baseline (speedup 1.0000x reference)
import functools

import jax
import jax.numpy as jnp
from jax import lax
from jax.experimental import pallas as pl
from jax.experimental.pallas import tpu as pltpu
from jax.experimental.pallas import tpu_sc as plsc

_NC = 2
_NS = 16
_NW = _NC * _NS


def _mv_body(t_ref, w_ref, o_ref):
    q, d = o_ref.shape
    t3 = t_ref[...].reshape(q, d, d)
    o_ref[...] = jnp.sum(t3 * w_ref[...][None], axis=-1)


def _table_matvec(table, w_row):
    V, D = table.shape
    Q = 512
    return pl.pallas_call(
        _mv_body,
        grid=(pl.cdiv(V, Q * D),),
        in_specs=[
            pl.BlockSpec((Q * D, D), lambda g: (g, 0)),
            pl.BlockSpec((1, D), lambda g: (0, 0)),
        ],
        out_specs=pl.BlockSpec((Q, D), lambda g: (g, 0)),
        out_shape=jax.ShapeDtypeStruct((V // D, D), jnp.float32),
    )(table, w_row)


@functools.cache
def _make_sc_bag(B, L):
    BW = B // _NW
    RW = B * L // (128 * _NW)
    mesh = plsc.VectorSubcoreMesh(core_axis_name="c", subcore_axis_name="s")

    @functools.partial(
        pl.kernel,
        out_type=jax.ShapeDtypeStruct((B,), jnp.float32),
        mesh=mesh,
        compiler_params=pltpu.CompilerParams(needs_layout_passes=False),
        scratch_types=[
            pltpu.VMEM((RW, 128), jnp.int32),
            pltpu.VMEM((RW, 128), jnp.float32),
            pltpu.VMEM((BW,), jnp.int32),
            pltpu.VMEM((16,), jnp.float32),
            pltpu.VMEM((BW,), jnp.float32),
            pltpu.SemaphoreType.DMA,
        ],
    )
    def sc_bag(x2_hbm, lens_hbm, tw_hbm, b16_hbm, out_hbm,
               idx_v, vals_v, lens_v, b_v, out_v, sem):
        wid = lax.axis_index("s") * _NC + lax.axis_index("c")
        pltpu.sync_copy(x2_hbm.at[pl.ds(wid * RW, RW)], idx_v)
        pltpu.sync_copy(lens_hbm.at[pl.ds(wid * BW, BW)], lens_v)
        pltpu.sync_copy(b16_hbm, b_v)
        K = 8

        def fire_body(c, carry):
            pltpu.async_copy(tw_hbm.at[idx_v.at[c]], vals_v.at[c], sem)

            @pl.when(c >= K)
            def _():
                d = c - K
                pltpu.make_async_copy(tw_hbm.at[idx_v.at[d]], vals_v.at[d], sem).wait()

            return carry

        lax.fori_loop(0, RW, fire_body, 0)

        def drain_body(c, carry):
            pltpu.make_async_copy(tw_hbm.at[idx_v.at[c]], vals_v.at[c], sem).wait()
            return carry

        lax.fori_loop(RW - K, RW, drain_body, 0)
        bias = b_v[...]
        for g in range(BW // 16):
            b_vec = g * 16 + lax.iota(jnp.int32, 16)
            lens_g = lens_v[pl.ds(g * 16, 16)]
            base = b_vec * L

            def body(j, acc, base=base, lens_g=lens_g):
                flat = base + j
                r = lax.shift_right_logical(flat, 7)
                cc = lax.bitwise_and(flat, 127)
                v = plsc.load_gather(vals_v, [r, cc])
                return acc + jnp.where(j < lens_g, v, 0.0)

            acc = lax.fori_loop(0, L, body, jnp.zeros((16,), jnp.float32))
            out_v[pl.ds(g * 16, 16)] = acc / lens_g.astype(jnp.float32) + bias
        pltpu.sync_copy(out_v, out_hbm.at[pl.ds(wid * BW, BW)])

    return sc_bag


def kernel(x, lens, table, W, b):
    B, L = x.shape
    V, D = table.shape
    tw = _table_matvec(table, W).reshape(V)
    x2 = x.reshape(-1, 128)
    b16 = jnp.broadcast_to(b.reshape(1).astype(jnp.float32), (16,))
    return _make_sc_bag(B, L)(x2, lens, tw, b16)


# speedup vs baseline: 1.1837x; 1.1837x over previous
import functools

import jax
import jax.numpy as jnp
from jax import lax
from jax.experimental import pallas as pl
from jax.experimental.pallas import tpu as pltpu
from jax.experimental.pallas import tpu_sc as plsc

_NC = 2
_NS = 16
_NW = _NC * _NS


def _mv_body(t_ref, w_ref, o_ref):
    q, d = o_ref.shape
    t3 = t_ref[...].reshape(q, d, d)
    o_ref[...] = jnp.sum(t3 * w_ref[...][None], axis=-1)


def _table_matvec(table, w_row):
    V, D = table.shape
    Q = 512
    return pl.pallas_call(
        _mv_body,
        grid=(pl.cdiv(V, Q * D),),
        in_specs=[
            pl.BlockSpec((Q * D, D), lambda g: (g, 0)),
            pl.BlockSpec((1, D), lambda g: (0, 0)),
        ],
        out_specs=pl.BlockSpec((Q, D), lambda g: (g, 0)),
        out_shape=jax.ShapeDtypeStruct((V // D, D), jnp.float32),
    )(table, w_row)


@functools.cache
def _make_sc_bag(B, L):
    BW = B // _NW
    RW = B * L // (128 * _NW)
    mesh = plsc.VectorSubcoreMesh(core_axis_name="c", subcore_axis_name="s")

    @functools.partial(
        pl.kernel,
        out_type=jax.ShapeDtypeStruct((B,), jnp.float32),
        mesh=mesh,
        compiler_params=pltpu.CompilerParams(needs_layout_passes=False),
        scratch_types=[
            pltpu.VMEM((RW, 128), jnp.int32),
            pltpu.VMEM((RW, 128), jnp.float32),
            pltpu.VMEM((BW,), jnp.int32),
            pltpu.VMEM((16,), jnp.float32),
            pltpu.VMEM((BW,), jnp.float32),
            pltpu.SemaphoreType.DMA,
        ],
    )
    def sc_bag(x2_hbm, lens_hbm, tw_hbm, b16_hbm, out_hbm,
               idx_v, vals_v, lens_v, b_v, out_v, sem):
        wid = lax.axis_index("s") * _NC + lax.axis_index("c")
        pltpu.sync_copy(x2_hbm.at[pl.ds(wid * RW, RW)], idx_v)
        pltpu.sync_copy(lens_hbm.at[pl.ds(wid * BW, BW)], lens_v)
        pltpu.sync_copy(b16_hbm, b_v)
        K = 8

        def fire_body(c, carry):
            pltpu.async_copy(tw_hbm.at[idx_v.at[c]], vals_v.at[c], sem)

            @pl.when(c >= K)
            def _():
                d = c - K
                pltpu.make_async_copy(tw_hbm.at[idx_v.at[d]], vals_v.at[d], sem).wait()

            return carry

        lax.fori_loop(0, RW, fire_body, 0)

        def drain_body(c, carry):
            pltpu.make_async_copy(tw_hbm.at[idx_v.at[c]], vals_v.at[c], sem).wait()
            return carry

        lax.fori_loop(RW - K, RW, drain_body, 0)
        bias = b_v[...]
        for g in range(BW // 16):
            b_vec = g * 16 + lax.iota(jnp.int32, 16)
            lens_g = lens_v[pl.ds(g * 16, 16)]
            base = b_vec * L

            def body(j, acc, base=base, lens_g=lens_g):
                flat = base + j
                r = lax.shift_right_logical(flat, 7)
                cc = lax.bitwise_and(flat, 127)
                v = plsc.load_gather(vals_v, [r, cc])
                return acc + jnp.where(j < lens_g, v, 0.0)

            acc = lax.fori_loop(0, L, body, jnp.zeros((16,), jnp.float32))
            out_v[pl.ds(g * 16, 16)] = acc / lens_g.astype(jnp.float32) + bias
        pltpu.sync_copy(out_v, out_hbm.at[pl.ds(wid * BW, BW)])

    return sc_bag


def kernel(x, lens, table, W, b):
    B, L = x.shape
    V, D = table.shape
    tw = _table_matvec(table, W).reshape(V)
    return tw[:B]
    x2 = x.reshape(-1, 128)
    b16 = jnp.broadcast_to(b.reshape(1).astype(jnp.float32), (16,))
    return _make_sc_bag(B, L)(x2, lens, tw, b16)


# speedup vs baseline: 3.7944x; 3.2054x over previous
import functools

import jax
import jax.numpy as jnp
from jax import lax
from jax.experimental import pallas as pl
from jax.experimental.pallas import tpu as pltpu
from jax.experimental.pallas import tpu_sc as plsc

_NC = 2
_NS = 16
_NW = _NC * _NS


def _mv_body(t_ref, w_ref, o_ref):
    o_ref[...] = jnp.sum(t_ref[...] * w_ref[...], axis=0)


def _table_matvec(tT, w_col):
    D, V = tT.shape
    CB = 65536
    return pl.pallas_call(
        _mv_body,
        grid=(pl.cdiv(V, CB),),
        in_specs=[
            pl.BlockSpec((D, CB), lambda g: (0, g)),
            pl.BlockSpec((D, 1), lambda g: (0, 0)),
        ],
        out_specs=pl.BlockSpec((CB,), lambda g: (g,)),
        out_shape=jax.ShapeDtypeStruct((V,), jnp.float32),
    )(tT, w_col)


@functools.cache
def _make_sc_bag(B, L):
    BW = B // _NW
    mesh = plsc.VectorSubcoreMesh(core_axis_name="c", subcore_axis_name="s")

    @functools.partial(
        pl.kernel,
        out_type=jax.ShapeDtypeStruct((B,), jnp.float32),
        mesh=mesh,
        compiler_params=pltpu.CompilerParams(needs_layout_passes=False),
        scratch_types=[
            pltpu.VMEM((L, BW), jnp.int32),
            pltpu.VMEM((L, BW), jnp.float32),
            pltpu.VMEM((BW,), jnp.int32),
            pltpu.VMEM((16,), jnp.float32),
            pltpu.VMEM((BW,), jnp.float32),
            pltpu.SemaphoreType.DMA,
        ],
    )
    def sc_bag(xT_hbm, lens_hbm, tw_hbm, b16_hbm, out_hbm,
               idx_v, vals_v, lens_v, b_v, out_v, sem):
        wid = lax.axis_index("s") * _NC + lax.axis_index("c")
        base = wid * BW
        pltpu.sync_copy(xT_hbm.at[:, pl.ds(base, BW)], idx_v)
        pltpu.sync_copy(lens_hbm.at[pl.ds(base, BW)], lens_v)
        pltpu.sync_copy(b16_hbm, b_v)
        K = 8

        def fire_body(c, carry):
            pltpu.async_copy(tw_hbm.at[idx_v.at[c]], vals_v.at[c], sem)

            @pl.when(c >= K)
            def _():
                d = c - K
                pltpu.make_async_copy(tw_hbm.at[idx_v.at[d]], vals_v.at[d], sem).wait()

            return carry

        lax.fori_loop(0, L, fire_body, 0)

        def drain_body(c, carry):
            pltpu.make_async_copy(tw_hbm.at[idx_v.at[c]], vals_v.at[c], sem).wait()
            return carry

        lax.fori_loop(L - K, L, drain_body, 0)

        bias = b_v[...]
        for g in range(BW // 16):
            sl = pl.ds(g * 16, 16)
            lens_g = lens_v[sl]

            def body(j, acc, sl=sl, lens_g=lens_g):
                v = vals_v[j, sl]
                return acc + jnp.where(j < lens_g, v, 0.0)

            acc = lax.fori_loop(0, L, body, jnp.zeros((16,), jnp.float32))
            out_v[sl] = acc / lens_g.astype(jnp.float32) + bias
        pltpu.sync_copy(out_v, out_hbm.at[pl.ds(base, BW)])

    return sc_bag


def kernel(x, lens, table, W, b):
    B, L = x.shape
    V, D = table.shape
    tw = _table_matvec(table.T, W.T)
    b16 = jnp.broadcast_to(b.reshape(1).astype(jnp.float32), (16,))
    return _make_sc_bag(B, L)(x.T, lens, tw, b16)


# speedup vs baseline: 4.2030x; 1.1077x over previous
import functools

import jax
import jax.numpy as jnp
from jax import lax
from jax.experimental import pallas as pl
from jax.experimental.pallas import tpu as pltpu
from jax.experimental.pallas import tpu_sc as plsc

_NC = 2
_NS = 16
_NW = _NC * _NS


def _mv_body(t_ref, w_ref, o_ref):
    o_ref[...] = jnp.sum(t_ref[...] * w_ref[...], axis=0)


def _table_matvec(tT, w_col):
    D, V = tT.shape
    CB = 65536
    return pl.pallas_call(
        _mv_body,
        grid=(pl.cdiv(V, CB),),
        in_specs=[
            pl.BlockSpec((D, CB), lambda g: (0, g)),
            pl.BlockSpec((D, 1), lambda g: (0, 0)),
        ],
        out_specs=pl.BlockSpec((CB,), lambda g: (g,)),
        out_shape=jax.ShapeDtypeStruct((V,), jnp.float32),
    )(tT, w_col)


@functools.cache
def _make_sc_bag(B, L):
    BW = B // _NW
    mesh = plsc.VectorSubcoreMesh(core_axis_name="c", subcore_axis_name="s")

    @functools.partial(
        pl.kernel,
        out_type=jax.ShapeDtypeStruct((B,), jnp.float32),
        mesh=mesh,
        compiler_params=pltpu.CompilerParams(needs_layout_passes=False),
        scratch_types=[
            pltpu.VMEM((L, BW), jnp.int32),
            pltpu.VMEM((L, BW), jnp.float32),
            pltpu.VMEM((BW,), jnp.int32),
            pltpu.VMEM((16,), jnp.float32),
            pltpu.VMEM((BW,), jnp.float32),
            pltpu.SemaphoreType.DMA((8,)),
        ],
    )
    def sc_bag(xT_hbm, lens_hbm, tw_hbm, b16_hbm, out_hbm,
               idx_v, vals_v, lens_v, b_v, out_v, sem):
        wid = lax.axis_index("s") * _NC + lax.axis_index("c")
        base = wid * BW
        pltpu.sync_copy(xT_hbm.at[:, pl.ds(base, BW)], idx_v)
        pltpu.sync_copy(lens_hbm.at[pl.ds(base, BW)], lens_v)
        pltpu.sync_copy(b16_hbm, b_v)
        K = 8
        NG = BW // 16
        lens_gs = [lens_v[pl.ds(g * 16, 16)] for g in range(NG)]

        def prime(c, carry):
            pltpu.async_copy(tw_hbm.at[idx_v.at[c]], vals_v.at[c], sem.at[c])
            return carry

        lax.fori_loop(0, K, prime, 0)

        def step(j, accs):
            @pl.when(j + K < L)
            def _():
                c = j + K
                pltpu.async_copy(tw_hbm.at[idx_v.at[c]], vals_v.at[c],
                                 sem.at[lax.rem(c, K)])

            pltpu.make_async_copy(tw_hbm.at[idx_v.at[j]], vals_v.at[j],
                                  sem.at[lax.rem(j, K)]).wait()
            out = []
            for g in range(NG):
                v = vals_v[j, pl.ds(g * 16, 16)]
                out.append(accs[g] + jnp.where(j < lens_gs[g], v, 0.0))
            return tuple(out)

        accs = lax.fori_loop(
            0, L, step, tuple(jnp.zeros((16,), jnp.float32) for _ in range(NG)))

        bias = b_v[...]
        for g in range(NG):
            sl = pl.ds(g * 16, 16)
            out_v[sl] = accs[g] / lens_gs[g].astype(jnp.float32) + bias
        pltpu.sync_copy(out_v, out_hbm.at[pl.ds(base, BW)])

    return sc_bag


def kernel(x, lens, table, W, b):
    B, L = x.shape
    V, D = table.shape
    tw = _table_matvec(table.T, W.T)
    b16 = jnp.broadcast_to(b.reshape(1).astype(jnp.float32), (16,))
    return _make_sc_bag(B, L)(x.T, lens, tw, b16)


# speedup vs baseline: 5.0936x; 1.2119x over previous
import functools

import jax
import jax.numpy as jnp
from jax import lax
from jax.experimental import pallas as pl
from jax.experimental.pallas import tpu as pltpu
from jax.experimental.pallas import tpu_sc as plsc

_NC = 2
_NS = 16
_NW = _NC * _NS


def _mv_body(t_ref, w_ref, o_ref):
    o_ref[...] = jnp.sum(t_ref[...] * w_ref[...], axis=0)


def _table_matvec(tT, w_col, v_pad):
    D, V = tT.shape
    CB = 65536
    return pl.pallas_call(
        _mv_body,
        grid=(pl.cdiv(v_pad, CB),),
        in_specs=[
            pl.BlockSpec((D, CB), lambda g: (0, g)),
            pl.BlockSpec((D, 1), lambda g: (0, 0)),
        ],
        out_specs=pl.BlockSpec((CB,), lambda g: (g,)),
        out_shape=jax.ShapeDtypeStruct((v_pad,), jnp.float32),
    )(tT, w_col)


@functools.cache
def _make_sc_bag(B, L, V):
    BW = B // _NW
    mesh = plsc.VectorSubcoreMesh(core_axis_name="c", subcore_axis_name="s")

    @functools.partial(
        pl.kernel,
        out_type=jax.ShapeDtypeStruct((B,), jnp.float32),
        mesh=mesh,
        compiler_params=pltpu.CompilerParams(needs_layout_passes=False),
        scratch_types=[
            pltpu.VMEM((L, BW), jnp.int32),
            pltpu.VMEM((L, BW), jnp.float32),
            pltpu.VMEM((BW,), jnp.int32),
            pltpu.VMEM((16,), jnp.float32),
            pltpu.VMEM((BW,), jnp.float32),
            pltpu.SemaphoreType.DMA((8,)),
            pltpu.VMEM_SHARED((V,), jnp.float32),
        ],
    )
    def sc_bag(xT_hbm, lens_hbm, tw_hbm, b16_hbm, out_hbm,
               idx_v, vals_v, lens_v, b_v, out_v, sem, tw_sh):
        sid = lax.axis_index("s")
        wid = sid * _NC + lax.axis_index("c")
        base = wid * BW
        pltpu.sync_copy(xT_hbm.at[:, pl.ds(base, BW)], idx_v)
        pltpu.sync_copy(lens_hbm.at[pl.ds(base, BW)], lens_v)
        pltpu.sync_copy(b16_hbm, b_v)
        CHUNK = 65536

        @pl.when(sid < _NS - 1)
        def _():
            pltpu.sync_copy(tw_hbm.at[pl.ds(sid * CHUNK, CHUNK)],
                            tw_sh.at[pl.ds(sid * CHUNK, CHUNK)])

        @pl.when(sid == _NS - 1)
        def _():
            rem = V - (_NS - 1) * CHUNK
            pltpu.sync_copy(tw_hbm.at[pl.ds((_NS - 1) * CHUNK, rem)],
                            tw_sh.at[pl.ds((_NS - 1) * CHUNK, rem)])

        plsc.subcore_barrier()
        K = 8
        NG = BW // 16
        lens_gs = [lens_v[pl.ds(g * 16, 16)] for g in range(NG)]

        def prime(c, carry):
            pltpu.async_copy(tw_sh.at[idx_v.at[c]], vals_v.at[c], sem.at[c])
            return carry

        lax.fori_loop(0, K, prime, 0)

        def step(j, accs):
            @pl.when(j + K < L)
            def _():
                c = j + K
                pltpu.async_copy(tw_sh.at[idx_v.at[c]], vals_v.at[c],
                                 sem.at[lax.rem(c, K)])

            pltpu.make_async_copy(tw_sh.at[idx_v.at[j]], vals_v.at[j],
                                  sem.at[lax.rem(j, K)]).wait()
            out = []
            for g in range(NG):
                v = vals_v[j, pl.ds(g * 16, 16)]
                out.append(accs[g] + jnp.where(j < lens_gs[g], v, 0.0))
            return tuple(out)

        accs = lax.fori_loop(
            0, L, step, tuple(jnp.zeros((16,), jnp.float32) for _ in range(NG)))

        bias = b_v[...]
        for g in range(NG):
            sl = pl.ds(g * 16, 16)
            out_v[sl] = accs[g] / lens_gs[g].astype(jnp.float32) + bias
        pltpu.sync_copy(out_v, out_hbm.at[pl.ds(base, BW)])

    return sc_bag


def kernel(x, lens, table, W, b):
    B, L = x.shape
    V, D = table.shape
    v_pad = ((V + 1023) // 1024) * 1024
    tw = _table_matvec(table.T, W.T, v_pad)
    b16 = jnp.broadcast_to(b.reshape(1).astype(jnp.float32), (16,))
    return _make_sc_bag(B, L, v_pad)(x.T, lens, tw, b16)


# speedup vs baseline: 5.1283x; 1.0068x over previous
import functools

import jax
import jax.numpy as jnp
from jax import lax
from jax.experimental import pallas as pl
from jax.experimental.pallas import tpu as pltpu
from jax.experimental.pallas import tpu_sc as plsc

_NC = 2
_NS = 16
_NW = _NC * _NS


def _mv_body(t_ref, w_ref, o_ref):
    o_ref[...] = jnp.sum(t_ref[...] * w_ref[...], axis=0)


def _table_matvec(tT, w_col, v_pad):
    D, V = tT.shape
    CB = 73728
    return pl.pallas_call(
        _mv_body,
        grid=(pl.cdiv(v_pad, CB),),
        in_specs=[
            pl.BlockSpec((D, CB), lambda g: (0, g)),
            pl.BlockSpec((D, 1), lambda g: (0, 0)),
        ],
        out_specs=pl.BlockSpec((CB,), lambda g: (g,)),
        out_shape=jax.ShapeDtypeStruct((v_pad,), jnp.float32),
    )(tT, w_col)


@functools.cache
def _make_sc_bag(B, L, V):
    BW = B // _NW
    mesh = plsc.VectorSubcoreMesh(core_axis_name="c", subcore_axis_name="s")

    @functools.partial(
        pl.kernel,
        out_type=jax.ShapeDtypeStruct((B,), jnp.float32),
        mesh=mesh,
        compiler_params=pltpu.CompilerParams(needs_layout_passes=False),
        scratch_types=[
            pltpu.VMEM((L, BW), jnp.int32),
            pltpu.VMEM((L, BW), jnp.float32),
            pltpu.VMEM((BW,), jnp.int32),
            pltpu.VMEM((16,), jnp.float32),
            pltpu.VMEM((BW,), jnp.float32),
            pltpu.SemaphoreType.DMA((8,)),
            pltpu.VMEM_SHARED((V,), jnp.float32),
        ],
    )
    def sc_bag(xT_hbm, lens_hbm, tw_hbm, b16_hbm, out_hbm,
               idx_v, vals_v, lens_v, b_v, out_v, sem, tw_sh):
        sid = lax.axis_index("s")
        wid = sid * _NC + lax.axis_index("c")
        base = wid * BW
        pltpu.sync_copy(xT_hbm.at[:, pl.ds(base, BW)], idx_v)
        pltpu.sync_copy(lens_hbm.at[pl.ds(base, BW)], lens_v)
        pltpu.sync_copy(b16_hbm, b_v)
        CHUNK = 65536

        @pl.when(sid < _NS - 1)
        def _():
            pltpu.sync_copy(tw_hbm.at[pl.ds(sid * CHUNK, CHUNK)],
                            tw_sh.at[pl.ds(sid * CHUNK, CHUNK)])

        @pl.when(sid == _NS - 1)
        def _():
            rem = V - (_NS - 1) * CHUNK
            pltpu.sync_copy(tw_hbm.at[pl.ds((_NS - 1) * CHUNK, rem)],
                            tw_sh.at[pl.ds((_NS - 1) * CHUNK, rem)])

        plsc.subcore_barrier()
        K = 8
        NG = BW // 16
        lens_gs = [lens_v[pl.ds(g * 16, 16)] for g in range(NG)]

        def prime(c, carry):
            pltpu.async_copy(tw_sh.at[idx_v.at[c]], vals_v.at[c], sem.at[c])
            return carry

        lax.fori_loop(0, K, prime, 0)

        def step(j, accs):
            @pl.when(j + K < L)
            def _():
                c = j + K
                pltpu.async_copy(tw_sh.at[idx_v.at[c]], vals_v.at[c],
                                 sem.at[lax.rem(c, K)])

            pltpu.make_async_copy(tw_sh.at[idx_v.at[j]], vals_v.at[j],
                                  sem.at[lax.rem(j, K)]).wait()
            out = []
            for g in range(NG):
                v = vals_v[j, pl.ds(g * 16, 16)]
                out.append(accs[g] + jnp.where(j < lens_gs[g], v, 0.0))
            return tuple(out)

        accs = lax.fori_loop(
            0, L, step, tuple(jnp.zeros((16,), jnp.float32) for _ in range(NG)))

        bias = b_v[...]
        for g in range(NG):
            sl = pl.ds(g * 16, 16)
            out_v[sl] = accs[g] / lens_gs[g].astype(jnp.float32) + bias
        pltpu.sync_copy(out_v, out_hbm.at[pl.ds(base, BW)])

    return sc_bag


def kernel(x, lens, table, W, b):
    B, L = x.shape
    V, D = table.shape
    v_pad = ((V + 1023) // 1024) * 1024
    tw = _table_matvec(table.T, W.T, v_pad)
    b16 = jnp.broadcast_to(b.reshape(1).astype(jnp.float32), (16,))
    return _make_sc_bag(B, L, v_pad)(x.T, lens, tw, b16)


# speedup vs baseline: 5.2494x; 1.0236x over previous
import functools

import jax
import jax.numpy as jnp
from jax import lax
from jax.experimental import pallas as pl
from jax.experimental.pallas import tpu as pltpu
from jax.experimental.pallas import tpu_sc as plsc

_NC = 2
_NS = 16
_NW = _NC * _NS


def _mv_body(t_ref, w_ref, o_ref):
    o_ref[...] = jnp.sum(t_ref[...] * w_ref[...], axis=0)


def _table_matvec(tT, w_col, v_pad):
    D, V = tT.shape
    CB = 32768
    return pl.pallas_call(
        _mv_body,
        grid=(pl.cdiv(v_pad, CB),),
        in_specs=[
            pl.BlockSpec((D, CB), lambda g: (0, g)),
            pl.BlockSpec((D, 1), lambda g: (0, 0)),
        ],
        out_specs=pl.BlockSpec((CB,), lambda g: (g,)),
        out_shape=jax.ShapeDtypeStruct((v_pad,), jnp.float32),
    )(tT, w_col)


@functools.cache
def _make_sc_bag(B, L, V):
    BW = B // _NW
    mesh = plsc.VectorSubcoreMesh(core_axis_name="c", subcore_axis_name="s")

    @functools.partial(
        pl.kernel,
        out_type=jax.ShapeDtypeStruct((B,), jnp.float32),
        mesh=mesh,
        compiler_params=pltpu.CompilerParams(needs_layout_passes=False),
        scratch_types=[
            pltpu.VMEM((L, BW), jnp.int32),
            pltpu.VMEM((L, BW), jnp.float32),
            pltpu.VMEM((BW,), jnp.int32),
            pltpu.VMEM((16,), jnp.float32),
            pltpu.VMEM((BW,), jnp.float32),
            pltpu.SemaphoreType.DMA((8,)),
            pltpu.VMEM_SHARED((V,), jnp.float32),
        ],
    )
    def sc_bag(xT_hbm, lens_hbm, tw_hbm, b16_hbm, out_hbm,
               idx_v, vals_v, lens_v, b_v, out_v, sem, tw_sh):
        sid = lax.axis_index("s")
        wid = sid * _NC + lax.axis_index("c")
        base = wid * BW
        pltpu.sync_copy(xT_hbm.at[:, pl.ds(base, BW)], idx_v)
        pltpu.sync_copy(lens_hbm.at[pl.ds(base, BW)], lens_v)
        pltpu.sync_copy(b16_hbm, b_v)
        CHUNK = 65536

        @pl.when(sid < _NS - 1)
        def _():
            pltpu.sync_copy(tw_hbm.at[pl.ds(sid * CHUNK, CHUNK)],
                            tw_sh.at[pl.ds(sid * CHUNK, CHUNK)])

        @pl.when(sid == _NS - 1)
        def _():
            rem = V - (_NS - 1) * CHUNK
            pltpu.sync_copy(tw_hbm.at[pl.ds((_NS - 1) * CHUNK, rem)],
                            tw_sh.at[pl.ds((_NS - 1) * CHUNK, rem)])

        plsc.subcore_barrier()
        K = 8
        NG = BW // 16
        lens_gs = [lens_v[pl.ds(g * 16, 16)] for g in range(NG)]

        def prime(c, carry):
            pltpu.async_copy(tw_sh.at[idx_v.at[c]], vals_v.at[c], sem.at[c])
            return carry

        lax.fori_loop(0, K, prime, 0)

        def step(j, accs):
            @pl.when(j + K < L)
            def _():
                c = j + K
                pltpu.async_copy(tw_sh.at[idx_v.at[c]], vals_v.at[c],
                                 sem.at[lax.rem(c, K)])

            pltpu.make_async_copy(tw_sh.at[idx_v.at[j]], vals_v.at[j],
                                  sem.at[lax.rem(j, K)]).wait()
            out = []
            for g in range(NG):
                v = vals_v[j, pl.ds(g * 16, 16)]
                out.append(accs[g] + jnp.where(j < lens_gs[g], v, 0.0))
            return tuple(out)

        accs = lax.fori_loop(
            0, L, step, tuple(jnp.zeros((16,), jnp.float32) for _ in range(NG)))

        bias = b_v[...]
        for g in range(NG):
            sl = pl.ds(g * 16, 16)
            out_v[sl] = accs[g] / lens_gs[g].astype(jnp.float32) + bias
        pltpu.sync_copy(out_v, out_hbm.at[pl.ds(base, BW)])

    return sc_bag


def kernel(x, lens, table, W, b):
    B, L = x.shape
    V, D = table.shape
    v_pad = ((V + 1023) // 1024) * 1024
    tw = _table_matvec(table.T, W.T, v_pad)
    b16 = jnp.broadcast_to(b.reshape(1).astype(jnp.float32), (16,))
    return _make_sc_bag(B, L, v_pad)(x.T, lens, tw, b16)
